# baseline (speedup 1.0000x reference)
import functools

import jax
import jax.numpy as jnp
from jax import lax
from jax.experimental import pallas as pl
from jax.experimental.pallas import tpu as pltpu
from jax.experimental.pallas import tpu_sc as plsc

B = 16384
D = 64
L = 16
NC = 2
NS = 16
NW = NC * NS
BPW = B // NW
G = BPW // L


def _gather_body(users_hbm, ut_hbm, out_hbm, uidx_v, urow_v, sem_u):
    wid = lax.axis_index("s") * NC + lax.axis_index("c")
    base = wid * BPW
    pltpu.sync_copy(users_hbm.at[pl.ds(base, BPW)], uidx_v)
    pltpu.async_copy(ut_hbm.at[uidx_v], urow_v, sem_u).wait()
    pltpu.sync_copy(urow_v, out_hbm.at[pl.ds(base, BPW)])


def _score_body(items_hbm, it_hbm, pop_hbm, uemb_hbm, out_hbm,
                iidx_v, irow_v, uemb_v, pop_v, out_v, sem_i, sem_p, sem_e):
    wid = lax.axis_index("s") * NC + lax.axis_index("c")
    base = wid * BPW
    pltpu.sync_copy(items_hbm.at[pl.ds(base, BPW)], iidx_v)
    ci = pltpu.async_copy(it_hbm.at[iidx_v], irow_v, sem_i)
    cp = pltpu.async_copy(pop_hbm.at[iidx_v], pop_v, sem_p)
    ce = pltpu.async_copy(uemb_hbm.at[pl.ds(base, BPW)], uemb_v, sem_e)
    ci.wait()
    cp.wait()
    ce.wait()

    lanes = lax.iota(jnp.int32, L)

    def g_body(g, _):
        row = g * L + lanes

        def d_body(d, accs):
            a0, a1 = accs
            c0 = jnp.full((L,), 2 * d, jnp.int32)
            c1 = c0 + 1
            u0 = plsc.load_gather(uemb_v, [row, c0])
            i0 = plsc.load_gather(irow_v, [row, c0])
            u1 = plsc.load_gather(uemb_v, [row, c1])
            i1 = plsc.load_gather(irow_v, [row, c1])
            return (a0 + u0 * i0, a1 + u1 * i1)

        zero = jnp.zeros((L,), jnp.float32)
        a0, a1 = lax.fori_loop(0, D // 2, d_body, (zero, zero))
        acc = a0 + a1
        r = jnp.where(acc > 0, acc + 1.0, jnp.exp(acc))
        p = pop_v[pl.ds(g * L, L)]
        out_v[pl.ds(g * L, L)] = r * p
        return 0

    lax.fori_loop(0, G, g_body, 0)
    pltpu.sync_copy(out_v, out_hbm.at[pl.ds(base, BPW)])


@functools.partial(jax.jit)
def _run(users, items, user_table, item_table, last_popularity):
    mesh = plsc.VectorSubcoreMesh(core_axis_name="c", subcore_axis_name="s")
    params = pltpu.CompilerParams(
        use_tc_tiling_on_sc=False, needs_layout_passes=False)
    gather = functools.partial(
        pl.kernel,
        mesh=mesh,
        out_type=jax.ShapeDtypeStruct((B, D), jnp.float32),
        scratch_types=[
            pltpu.VMEM((BPW,), jnp.int32),
            pltpu.VMEM((BPW, D), jnp.float32),
            pltpu.SemaphoreType.DMA,
        ],
        compiler_params=params,
    )(_gather_body)
    score = functools.partial(
        pl.kernel,
        mesh=mesh,
        out_type=jax.ShapeDtypeStruct((B,), jnp.float32),
        scratch_types=[
            pltpu.VMEM((BPW,), jnp.int32),
            pltpu.VMEM((BPW, D), jnp.float32),
            pltpu.VMEM((BPW, D), jnp.float32),
            pltpu.VMEM((BPW,), jnp.float32),
            pltpu.VMEM((BPW,), jnp.float32),
            pltpu.SemaphoreType.DMA,
            pltpu.SemaphoreType.DMA,
            pltpu.SemaphoreType.DMA,
        ],
        compiler_params=params,
    )(_score_body)
    uemb = gather(users, user_table)
    return score(items, item_table, last_popularity, uemb)


def kernel(users, items, user_table, item_table, last_popularity):
    return _run(users.astype(jnp.int32), items.astype(jnp.int32),
                user_table, item_table, last_popularity)
